# initial kernel scaffold (unmeasured)
import jax
import jax.numpy as jnp
from jax import lax
from jax.experimental import pallas as pl
from jax.experimental.pallas import tpu as pltpu

NZ = 4
N_LOCAL_E = 2


def kernel(x, assign, W1, W2):
    t, d = x.shape
    n_e, _, f = W1.shape
    assign2 = assign.reshape(t, 1)

    def body(x_ref, a_ref, w1_ref, w2_ref, out_ref,
             xcomm, acomm, pbuf, psend_buf,
             xsend_sems, xrecv_sems, asend_sems, arecv_sems,
             psend_sems, precv_sems):
        my_x = lax.axis_index("x")
        my_y = lax.axis_index("y")
        my_z = lax.axis_index("z")
        right = (my_z + 1) % NZ

        barrier_sem = pltpu.get_barrier_semaphore()
        for dz in range(1, NZ):
            pl.semaphore_signal(
                barrier_sem, inc=1,
                device_id=(my_x, my_y, (my_z + dz) % NZ),
                device_id_type=pl.DeviceIdType.MESH,
            )
        pl.semaphore_wait(barrier_sem, NZ - 1)

        xcomm[0, :, :] = x_ref[:, :]
        acomm[0, :, :] = a_ref[:, :]

        def start_hop(s):
            rx = pltpu.make_async_remote_copy(
                src_ref=xcomm.at[s], dst_ref=xcomm.at[s + 1],
                send_sem=xsend_sems.at[s], recv_sem=xrecv_sems.at[s],
                device_id=(my_x, my_y, right),
                device_id_type=pl.DeviceIdType.MESH,
            )
            ra = pltpu.make_async_remote_copy(
                src_ref=acomm.at[s], dst_ref=acomm.at[s + 1],
                send_sem=asend_sems.at[s], recv_sem=arecv_sems.at(s) if False else arecv_sems.at[s],
                device_id=(my_x, my_y, right),
                device_id_type=pl.DeviceIdType.MESH,
            )
            rx.start()
            ra.start()
            return rx, ra

        def ffn(s):
            xc = xcomm[s]
            ac = acomm[s]
            acc = None
            for el in range(N_LOCAL_E):
                e = my_z * N_LOCAL_E + el
                mask = (ac == e).astype(jnp.float32)
                xm = xc * mask
                h = jnp.maximum(
                    jnp.dot(xm, w1_ref[el], preferred_element_type=jnp.float32),
                    0.0,
                )
                p = jnp.dot(h, w2_ref[el], preferred_element_type=jnp.float32)
                acc = p if acc is None else acc + p
            return acc

        hops = []
        hops.append(start_hop(0))
        own = ffn(0)

        partial_rdmas = []
        for s in range(1, NZ):
            rx, ra = hops[s - 1]
            rx.wait_recv()
            ra.wait_recv()
            if s < NZ - 1:
                hops.append(start_hop(s))
            psend_buf[s - 1, :, :] = ffn(s)
            owner = (my_z - s) % NZ
            rp = pltpu.make_async_remote_copy(
                src_ref=psend_buf.at[s - 1], dst_ref=pbuf.at[s - 1],
                send_sem=psend_sems.at[s - 1], recv_sem=precv_sems.at[s - 1],
                device_id=(my_x, my_y, owner),
                device_id_type=pl.DeviceIdType.MESH,
            )
            rp.start()
            partial_rdmas.append(rp)

        for rp in partial_rdmas:
            rp.wait_recv()
        out_ref[:, :] = own + pbuf[0] + pbuf[1] + pbuf[2]

        for rx, ra in hops:
            rx.wait_send()
            ra.wait_send()
        for rp in partial_rdmas:
            rp.wait_send()

    return pl.pallas_call(
        body,
        out_shape=jax.ShapeDtypeStruct((t, d), jnp.float32),
        in_specs=[
            pl.BlockSpec(memory_space=pltpu.VMEM),
            pl.BlockSpec(memory_space=pltpu.VMEM),
            pl.BlockSpec(memory_space=pltpu.VMEM),
            pl.BlockSpec(memory_space=pltpu.VMEM),
        ],
        out_specs=pl.BlockSpec(memory_space=pltpu.VMEM),
        scratch_shapes=[
            pltpu.VMEM((NZ, t, d), jnp.float32),
            pltpu.VMEM((NZ, t, 1), jnp.int32),
            pltpu.VMEM((NZ - 1, t, d), jnp.float32),
            pltpu.VMEM((NZ - 1, t, d), jnp.float32),
            pltpu.SemaphoreType.DMA((NZ - 1,)),
            pltpu.SemaphoreType.DMA((NZ - 1,)),
            pltpu.SemaphoreType.DMA((NZ - 1,)),
            pltpu.SemaphoreType.DMA((NZ - 1,)),
            pltpu.SemaphoreType.DMA((NZ - 1,)),
            pltpu.SemaphoreType.DMA((NZ - 1,)),
        ],
        compiler_params=pltpu.CompilerParams(collective_id=0),
    )(x, assign2, W1, W2)


# baseline (device time: 351311 ns/iter reference)
import jax
import jax.numpy as jnp
from jax import lax
from jax.experimental import pallas as pl
from jax.experimental.pallas import tpu as pltpu

NZ = 4
N_LOCAL_E = 2
FB = 512
TB = 256


def kernel(x, assign, W1, W2):
    t, d = x.shape
    n_e, _, f = W1.shape
    assign2 = assign.reshape(t, 1)
    n_fb = f // FB

    def body(x_ref, a_ref, w1_ref, w2_ref, out_ref,
             xcomm, acomm, pbuf, psend_buf, w1_stage, w2_stage,
             xsend_sems, xrecv_sems, asend_sems, arecv_sems,
             psend_sems, precv_sems, w_sems):
        my_x = lax.axis_index("x")
        my_y = lax.axis_index("y")
        my_z = lax.axis_index("z")
        right = (my_z + 1) % NZ

        barrier_sem = pltpu.get_barrier_semaphore()
        for dz in range(1, NZ):
            pl.semaphore_signal(
                barrier_sem, inc=1,
                device_id=(my_x, my_y, (my_z + dz) % NZ),
                device_id_type=pl.DeviceIdType.MESH,
            )
        pl.semaphore_wait(barrier_sem, NZ - 1)

        def start_hop(s):
            xsrc = x_ref if s == 0 else xcomm.at[s - 1]
            asrc = a_ref if s == 0 else acomm.at[s - 1]
            rx = pltpu.make_async_remote_copy(
                src_ref=xsrc, dst_ref=xcomm.at[s],
                send_sem=xsend_sems.at[s], recv_sem=xrecv_sems.at[s],
                device_id=(my_x, my_y, right),
                device_id_type=pl.DeviceIdType.MESH,
            )
            ra = pltpu.make_async_remote_copy(
                src_ref=asrc, dst_ref=acomm.at[s],
                send_sem=asend_sems.at[s], recv_sem=arecv_sems.at[s],
                device_id=(my_x, my_y, right),
                device_id_type=pl.DeviceIdType.MESH,
            )
            rx.start()
            ra.start()
            return rx, ra

        def ffn_into(dst_ref, x_src, a_src):
            dst_ref[:, :] = jnp.zeros((t, d), jnp.float32)
            for el in range(N_LOCAL_E):
                e = my_z * N_LOCAL_E + el
                for fb in range(n_fb):
                    c1 = pltpu.make_async_copy(
                        w1_ref.at[el, :, pl.ds(fb * FB, FB)],
                        w1_stage, w_sems.at[0])
                    c2 = pltpu.make_async_copy(
                        w2_ref.at[el, pl.ds(fb * FB, FB), :],
                        w2_stage, w_sems.at[1])
                    c1.start()
                    c2.start()
                    c1.wait()
                    c2.wait()
                    for tb in range(t // TB):
                        rows = pl.ds(tb * TB, TB)
                        mask = (a_src[rows, :] == e).astype(jnp.float32)
                        xm = x_src[rows, :] * mask
                        h = jnp.maximum(
                            jnp.dot(xm, w1_stage[:, :],
                                    preferred_element_type=jnp.float32),
                            0.0,
                        )
                        dst_ref[rows, :] = dst_ref[rows, :] + jnp.dot(
                            h, w2_stage[:, :],
                            preferred_element_type=jnp.float32)

        hops = [start_hop(0)]
        ffn_into(out_ref, x_ref, a_ref)

        partial_rdmas = []
        for s in range(NZ - 1):
            rx, ra = hops[s]
            rx.wait_recv()
            ra.wait_recv()
            if s < NZ - 2:
                hops.append(start_hop(s + 1))
            slot = s % 2
            if s >= 2:
                partial_rdmas[s - 2].wait_send()
            ffn_into(psend_buf.at[slot], xcomm.at[s], acomm.at[s])
            owner = (my_z - 1 - s) % NZ
            rp = pltpu.make_async_remote_copy(
                src_ref=psend_buf.at[slot], dst_ref=pbuf.at[s],
                send_sem=psend_sems.at[slot], recv_sem=precv_sems.at[s],
                device_id=(my_x, my_y, owner),
                device_id_type=pl.DeviceIdType.MESH,
            )
            rp.start()
            partial_rdmas.append(rp)

        for rp in partial_rdmas:
            rp.wait_recv()
        for s in range(NZ - 1):
            out_ref[:, :] = out_ref[:, :] + pbuf[s]

        for rx, ra in hops:
            rx.wait_send()
            ra.wait_send()
        for rp in partial_rdmas[1:]:
            rp.wait_send()

    return pl.pallas_call(
        body,
        out_shape=jax.ShapeDtypeStruct((t, d), jnp.float32),
        in_specs=[
            pl.BlockSpec(memory_space=pltpu.VMEM),
            pl.BlockSpec(memory_space=pltpu.VMEM),
            pl.BlockSpec(memory_space=pl.ANY),
            pl.BlockSpec(memory_space=pl.ANY),
        ],
        out_specs=pl.BlockSpec(memory_space=pltpu.VMEM),
        scratch_shapes=[
            pltpu.VMEM((NZ - 1, t, d), jnp.float32),
            pltpu.VMEM((NZ - 1, t, 1), jnp.int32),
            pltpu.VMEM((NZ - 1, t, d), jnp.float32),
            pltpu.VMEM((2, t, d), jnp.float32),
            pltpu.VMEM((d, FB), jnp.float32),
            pltpu.VMEM((FB, d), jnp.float32),
            pltpu.SemaphoreType.DMA((NZ - 1,)),
            pltpu.SemaphoreType.DMA((NZ - 1,)),
            pltpu.SemaphoreType.DMA((NZ - 1,)),
            pltpu.SemaphoreType.DMA((NZ - 1,)),
            pltpu.SemaphoreType.DMA((2,)),
            pltpu.SemaphoreType.DMA((NZ - 1,)),
            pltpu.SemaphoreType.DMA((2,)),
        ],
        compiler_params=pltpu.CompilerParams(
            collective_id=0,
            vmem_limit_bytes=60 * 1024 * 1024,
        ),
    )(x, assign2, W1, W2)
